# baseline (speedup 1.0000x reference)
import functools

import jax
import jax.numpy as jnp
from jax import lax
from jax.experimental import pallas as pl
from jax.experimental.pallas import tpu as pltpu
from jax.experimental.pallas import tpu_sc as plsc

_B, _NP, _H, _W = 1024, 32, 64, 64
_LANES = _W * _NP
_NC, _NS = 2, 16
_NWORK = _NC * _NS
_BPW = _B // _NWORK
_CR = 16
_CH = _H // _CR
_CHW = _CR * _LANES
_SLAB = _H * _LANES


def _sc_raster(x_hbm, res_hbm, org_hbm, out_hbm, x_v, res_v, org_v, buf):
    wid = lax.axis_index("s") * _NC + lax.axis_index("c")
    base = wid * _BPW

    pltpu.sync_copy(x_hbm.at[pl.ds(base * 2 * _NP, _BPW * 2 * _NP)], x_v)
    pltpu.sync_copy(res_hbm, res_v)
    pltpu.sync_copy(org_hbm, org_v)

    zeros16f = jnp.zeros((16,), jnp.float32)

    def _zero(j, carry):
        buf[pl.ds(j * 16, 16)] = zeros16f
        return carry

    lax.fori_loop(0, _CHW // 16, _zero, None)

    iota = lax.iota(jnp.int32, 16)
    ones16f = jnp.ones((16,), jnp.float32)

    def _batch(b, carry):
        gb = base + b
        bvec = jnp.full((16,), b * 2 * _NP, jnp.int32)
        gvec = jnp.full((16,), 2 * gb, jnp.int32)
        res0 = plsc.load_gather(res_v, [gvec])
        res1 = plsc.load_gather(res_v, [gvec + 1])
        org0 = plsc.load_gather(org_v, [gvec])
        org1 = plsc.load_gather(org_v, [gvec + 1])
        rows = []
        kpos = []
        okc = []
        for h in range(2):
            c = iota + (16 * h)
            px = plsc.load_gather(x_v, [bvec + 2 * c])
            py = plsc.load_gather(x_v, [bvec + 2 * c + 1])
            row = (py / res0 + org0).astype(jnp.int32)
            col = (px / res1 + org1).astype(jnp.int32)
            rows.append(row)
            kpos.append(col * _NP + c)
            okc.append((col >= 0) & (col < _W))
        for q in range(_CH):
            lo = q * _CR
            ms = []
            fidx = []
            for h in range(2):
                m = (rows[h] >= lo) & (rows[h] < lo + _CR) & okc[h]
                f = (rows[h] - lo) * _LANES + kpos[h]
                fidx.append(jnp.where(m, f, 0))
                ms.append(m)
                plsc.store_scatter(buf, [fidx[h]], ones16f, mask=m)
            pltpu.sync_copy(
                buf, out_hbm.at[pl.ds(gb * _SLAB + lo * _LANES, _CHW)]
            )
            for h in range(2):
                plsc.store_scatter(buf, [fidx[h]], zeros16f, mask=ms[h])
        return carry

    lax.fori_loop(0, _BPW, _batch, None)


def kernel(x, resolution, origin):
    mesh = plsc.VectorSubcoreMesh(core_axis_name="c", subcore_axis_name="s")
    fn = functools.partial(
        pl.kernel,
        mesh=mesh,
        out_type=jax.ShapeDtypeStruct((_B * _H * _LANES,), jnp.float32),
        scratch_types=[
            pltpu.VMEM((_BPW * 2 * _NP,), jnp.float32),
            pltpu.VMEM((_B * 2,), jnp.float32),
            pltpu.VMEM((_B * 2,), jnp.float32),
            pltpu.VMEM((_CHW,), jnp.float32),
        ],
        compiler_params=pltpu.CompilerParams(needs_layout_passes=False),
    )(_sc_raster)
    out_flat = fn(x.reshape(-1), resolution.reshape(-1), origin.reshape(-1))
    return out_flat.reshape(_B, _H, _W, _NP)
